# Initial kernel scaffold; baseline (speedup 1.0000x reference)
import jax
import jax.numpy as jnp
from jax.experimental import pallas as pl


def kernel(preds, labels, mask, adj_matrix):
    raise NotImplementedError("write your pallas kernel here")


# speedup vs baseline: 4.2388x; 4.2388x over previous
import math

import jax
import jax.numpy as jnp
from jax.experimental import pallas as pl
from jax.experimental.pallas import tpu as pltpu

_N = 2048
_C = 4
_BP = 512
_BQ = 512
_PER = 0.001
_SIG1 = 1.0 / (1.0 + math.exp(-1.0))


def _eloss_kernel(a_p_ref, a_q_ref, preds_p_ref, preds_q_ref,
                  lab_p_ref, lab_q_ref, msk_p_ref, msk_q_ref,
                  out_ref, r_acc, sub_acc, int_acc, nvec_acc, ce_acc):
    pb = pl.program_id(0)
    qb = pl.program_id(1)
    npb = pl.num_programs(0)
    nqb = pl.num_programs(1)

    @pl.when(jnp.logical_and(pb == 0, qb == 0))
    def _init():
        r_acc[...] = jnp.zeros_like(r_acc)
        sub_acc[...] = jnp.zeros_like(sub_acc)
        int_acc[...] = jnp.zeros_like(int_acc)
        nvec_acc[...] = jnp.zeros_like(nvec_acc)
        ce_acc[...] = jnp.zeros_like(ce_acc)

    a_p = a_p_ref[...]
    a_q = a_q_ref[...]

    cnt_int = jax.lax.dot_general(
        a_p, a_q, (((1,), (1,)), ((), ())),
        preferred_element_type=jnp.float32)

    deg_p = jnp.sum(a_p.astype(jnp.float32), axis=1, keepdims=True)
    a_pq = a_p_ref[:, pl.ds(qb * _BQ, _BQ)].astype(jnp.float32)
    a_qq = a_q_ref[:, pl.ds(qb * _BQ, _BQ)].astype(jnp.float32)
    ir = jax.lax.broadcasted_iota(jnp.int32, (_BQ, _BQ), 0)
    ic = jax.lax.broadcasted_iota(jnp.int32, (_BQ, _BQ), 1)
    eye_q = (ir == ic).astype(jnp.float32)
    diag_q = jnp.sum(a_qq * eye_q, axis=0, keepdims=True)

    cnt_sub = deg_p - cnt_int - a_pq * (1.0 - diag_q)

    ratio = (1.0 + _SIG1 * cnt_sub) / (1.0 + _SIG1 * cnt_int)
    v = 1.0 - 1.0 / (1.0 + jnp.exp(-ratio))

    preds_p = preds_p_ref[...]
    preds_q = preds_q_ref[...]
    cls_p = jax.lax.broadcasted_iota(jnp.int32, (_BP, _C), 1)
    cls_q = jax.lax.broadcasted_iota(jnp.int32, (_BQ, _C), 1)
    poh = (lab_p_ref[...] == cls_p).astype(jnp.float32) * msk_p_ref[...]
    qoh = (lab_q_ref[...] == cls_q).astype(jnp.float32) * msk_q_ref[...]

    x_exp = poh * jnp.exp(-preds_p)
    e_q = jnp.exp(preds_q)
    y = jnp.concatenate([e_q[:, i:i + 1] * qoh for i in range(_C)], axis=1)

    m1 = jnp.dot(v, y, preferred_element_type=jnp.float32)
    r_blk = jax.lax.dot_general(
        x_exp, m1, (((0,), (0,)), ((), ())),
        preferred_element_type=jnp.float32)
    r_acc[...] += r_blk

    sub_pos = (cnt_sub > 0.0).astype(jnp.float32)
    int_pos = (cnt_int > 0.0).astype(jnp.float32)
    s1 = jnp.dot(sub_pos, qoh, preferred_element_type=jnp.float32)
    i1 = jnp.dot(int_pos, qoh, preferred_element_type=jnp.float32)
    sub_acc[...] += jax.lax.dot_general(
        poh, s1, (((0,), (0,)), ((), ())), preferred_element_type=jnp.float32)
    int_acc[...] += jax.lax.dot_general(
        poh, i1, (((0,), (0,)), ((), ())), preferred_element_type=jnp.float32)

    @pl.when(qb == 0)
    def _row_stats():
        nvec_acc[...] += jnp.sum(poh, axis=0, keepdims=True)
        m = jnp.max(preds_p, axis=1, keepdims=True)
        lse = m + jnp.log(jnp.sum(jnp.exp(preds_p - m), axis=1, keepdims=True))
        oh = (lab_p_ref[...] == cls_p).astype(jnp.float32)
        pick = jnp.sum(oh * preds_p, axis=1, keepdims=True)
        ce_acc[...] += jnp.sum(lse - pick)

    @pl.when(jnp.logical_and(pb == npb - 1, qb == nqb - 1))
    def _final():
        nv = nvec_acc[...]
        denom = jax.lax.dot_general(
            nv, nv, (((0,), (0,)), ((), ())),
            preferred_element_type=jnp.float32)
        inv = 1.0 / jnp.where(denom > 0.0, denom, 1.0)
        cond = jnp.logical_and(sub_acc[...] > 0.0, int_acc[...] > 0.0)
        pair = jnp.concatenate(
            [r_acc[i:i + 1, 4 * i:4 * i + 4] for i in range(_C)], axis=0)
        i4r = jax.lax.broadcasted_iota(jnp.int32, (_C, _C), 0)
        i4c = jax.lax.broadcasted_iota(jnp.int32, (_C, _C), 1)
        offdiag = i4r != i4c
        contrib = jnp.where(jnp.logical_and(cond, offdiag), pair * inv, 0.0)
        out_ref[...] = ce_acc[...] / float(_N) + _PER * jnp.sum(contrib)


def kernel(preds, labels, mask, adj_matrix):
    a_bf = adj_matrix.astype(jnp.bfloat16)
    lab2 = labels.reshape(_N, 1).astype(jnp.int32)
    msk2 = mask.reshape(_N, 1).astype(jnp.float32)

    out = pl.pallas_call(
        _eloss_kernel,
        grid=(_N // _BP, _N // _BQ),
        in_specs=[
            pl.BlockSpec((_BP, _N), lambda pb, qb: (pb, 0)),
            pl.BlockSpec((_BQ, _N), lambda pb, qb: (qb, 0)),
            pl.BlockSpec((_BP, _C), lambda pb, qb: (pb, 0)),
            pl.BlockSpec((_BQ, _C), lambda pb, qb: (qb, 0)),
            pl.BlockSpec((_BP, 1), lambda pb, qb: (pb, 0)),
            pl.BlockSpec((_BQ, 1), lambda pb, qb: (qb, 0)),
            pl.BlockSpec((_BP, 1), lambda pb, qb: (pb, 0)),
            pl.BlockSpec((_BQ, 1), lambda pb, qb: (qb, 0)),
        ],
        out_specs=pl.BlockSpec((1, 1), lambda pb, qb: (0, 0)),
        out_shape=jax.ShapeDtypeStruct((1, 1), jnp.float32),
        scratch_shapes=[
            pltpu.VMEM((_C, 4 * _C), jnp.float32),
            pltpu.VMEM((_C, _C), jnp.float32),
            pltpu.VMEM((_C, _C), jnp.float32),
            pltpu.VMEM((1, _C), jnp.float32),
            pltpu.VMEM((1, 1), jnp.float32),
        ],
    )(a_bf, a_bf, preds, preds, lab2, lab2, msk2, msk2)
    return out.reshape(())


# speedup vs baseline: 4.4839x; 1.0578x over previous
import math

import jax
import jax.numpy as jnp
from jax.experimental import pallas as pl
from jax.experimental.pallas import tpu as pltpu

_N = 2048
_C = 4
_BP = 512
_BQ = 512
_PER = 0.001
_SIG1 = 1.0 / (1.0 + math.exp(-1.0))


def _eloss_kernel(a_p_ref, a_q_ref, preds_q_ref, lab_q_ref, msk_q_ref,
                  out_ref,
                  degs_all, odq_all, qoh_all, y_all, xp8_all,
                  rhs_acc, acc24, nvec_acc, ce_acc):
    pb = pl.program_id(0)
    qb = pl.program_id(1)
    npb = pl.num_programs(0)
    nqb = pl.num_programs(1)

    @pl.when(jnp.logical_and(pb == 0, qb == 0))
    def _init():
        rhs_acc[...] = jnp.zeros_like(rhs_acc)
        acc24[...] = jnp.zeros_like(acc24)
        nvec_acc[...] = jnp.zeros_like(nvec_acc)
        ce_acc[...] = jnp.zeros_like(ce_acc)

    @pl.when(pb == 0)
    def _precompute():
        a_q = a_q_ref[...]
        degs_all[pl.ds(qb * _BQ, _BQ), :] = jnp.sum(
            a_q.astype(jnp.float32), axis=1, keepdims=True)

        a_qq = a_q_ref[:, pl.ds(qb * _BQ, _BQ)].astype(jnp.float32)
        ir = jax.lax.broadcasted_iota(jnp.int32, (_BQ, _BQ), 0)
        ic = jax.lax.broadcasted_iota(jnp.int32, (_BQ, _BQ), 1)
        diag_q = jnp.sum(a_qq * (ir == ic).astype(jnp.float32),
                         axis=0, keepdims=True)
        odq_all[:, pl.ds(qb * _BQ, _BQ)] = 1.0 - diag_q

        preds_q = preds_q_ref[...]
        cls_q = jax.lax.broadcasted_iota(jnp.int32, (_BQ, _C), 1)
        oh = (lab_q_ref[...] == cls_q).astype(jnp.float32)
        qoh = oh * msk_q_ref[...]
        qoh_all[pl.ds(qb * _BQ, _BQ), :] = qoh

        e_q = jnp.exp(preds_q)
        y_all[pl.ds(qb * _BQ, _BQ), :] = jnp.concatenate(
            [e_q[:, i:i + 1] * qoh for i in range(_C)], axis=1)
        xp8_all[pl.ds(qb * _BQ, _BQ), :] = jnp.concatenate(
            [qoh * jnp.exp(-preds_q), qoh], axis=1)

        nvec_acc[...] += jnp.sum(qoh, axis=0, keepdims=True)
        m = jnp.max(preds_q, axis=1, keepdims=True)
        lse = m + jnp.log(jnp.sum(jnp.exp(preds_q - m), axis=1, keepdims=True))
        pick = jnp.sum(oh * preds_q, axis=1, keepdims=True)
        ce_acc[...] += jnp.sum(lse - pick)

    a_p = a_p_ref[...]
    a_q = a_q_ref[...]
    cnt_int = jax.lax.dot_general(
        a_p, a_q, (((1,), (1,)), ((), ())),
        preferred_element_type=jnp.float32)

    dp = degs_all[pl.ds(pb * _BP, _BP), :]
    odq = odq_all[:, pl.ds(qb * _BQ, _BQ)]
    a_pq = a_p_ref[:, pl.ds(qb * _BQ, _BQ)].astype(jnp.float32)
    cnt_sub = dp - cnt_int - a_pq * odq

    ind_sub = jnp.minimum(cnt_sub, 1.0)
    ind_int = jnp.minimum(cnt_int, 1.0)
    ratio = (1.0 + _SIG1 * cnt_sub) / (1.0 + _SIG1 * cnt_int)
    t = jnp.exp(-ratio)
    v = t / (1.0 + t)

    y_q = y_all[pl.ds(qb * _BQ, _BQ), :]
    qoh_q = qoh_all[pl.ds(qb * _BQ, _BQ), :]
    m1 = jnp.dot(v, y_q, preferred_element_type=jnp.float32)
    s1 = jnp.dot(ind_sub, qoh_q, preferred_element_type=jnp.float32)
    i1 = jnp.dot(ind_int, qoh_q, preferred_element_type=jnp.float32)
    rhs_acc[...] += jnp.concatenate([m1, s1, i1], axis=1)

    @pl.when(qb == nqb - 1)
    def _row_reduce():
        lhs8 = xp8_all[pl.ds(pb * _BP, _BP), :]
        acc24[...] += jax.lax.dot_general(
            lhs8, rhs_acc[...], (((0,), (0,)), ((), ())),
            preferred_element_type=jnp.float32)
        rhs_acc[...] = jnp.zeros_like(rhs_acc)

    @pl.when(jnp.logical_and(pb == npb - 1, qb == nqb - 1))
    def _final():
        nv = nvec_acc[...]
        denom = jax.lax.dot_general(
            nv, nv, (((0,), (0,)), ((), ())),
            preferred_element_type=jnp.float32)
        inv = 1.0 / jnp.where(denom > 0.0, denom, 1.0)
        cond = jnp.logical_and(acc24[4:8, 16:20] > 0.0,
                               acc24[4:8, 20:24] > 0.0)
        pair = jnp.concatenate(
            [acc24[i:i + 1, 4 * i:4 * i + 4] for i in range(_C)], axis=0)
        i4r = jax.lax.broadcasted_iota(jnp.int32, (_C, _C), 0)
        i4c = jax.lax.broadcasted_iota(jnp.int32, (_C, _C), 1)
        offdiag = i4r != i4c
        contrib = jnp.where(jnp.logical_and(cond, offdiag), pair * inv, 0.0)
        out_ref[...] = ce_acc[...] / float(_N) + _PER * jnp.sum(contrib)


def kernel(preds, labels, mask, adj_matrix):
    a_bf = adj_matrix.astype(jnp.bfloat16)
    lab2 = labels.reshape(_N, 1).astype(jnp.int32)
    msk2 = mask.reshape(_N, 1).astype(jnp.float32)

    out = pl.pallas_call(
        _eloss_kernel,
        grid=(_N // _BP, _N // _BQ),
        in_specs=[
            pl.BlockSpec((_BP, _N), lambda pb, qb: (pb, 0)),
            pl.BlockSpec((_BQ, _N), lambda pb, qb: (qb, 0)),
            pl.BlockSpec((_BQ, _C), lambda pb, qb: (qb, 0)),
            pl.BlockSpec((_BQ, 1), lambda pb, qb: (qb, 0)),
            pl.BlockSpec((_BQ, 1), lambda pb, qb: (qb, 0)),
        ],
        out_specs=pl.BlockSpec((1, 1), lambda pb, qb: (0, 0)),
        out_shape=jax.ShapeDtypeStruct((1, 1), jnp.float32),
        scratch_shapes=[
            pltpu.VMEM((_N, 1), jnp.float32),
            pltpu.VMEM((1, _N), jnp.float32),
            pltpu.VMEM((_N, _C), jnp.float32),
            pltpu.VMEM((_N, 4 * _C), jnp.float32),
            pltpu.VMEM((_N, 2 * _C), jnp.float32),
            pltpu.VMEM((_BP, 6 * _C), jnp.float32),
            pltpu.VMEM((2 * _C, 6 * _C), jnp.float32),
            pltpu.VMEM((1, _C), jnp.float32),
            pltpu.VMEM((1, 1), jnp.float32),
        ],
    )(a_bf, a_bf, preds, lab2, msk2)
    return out.reshape(())
